# baseline (speedup 1.0000x reference)
import functools

import jax
import jax.numpy as jnp
from jax.experimental import pallas as pl
from jax.experimental.pallas import tpu as pltpu

_TILE_LANES = 21888


def _pf_t_kernel(xt_ref, w_ref, u_ref, b_ref, o_ref):
    z = xt_ref[...]
    s = jax.lax.dot_general(
        w_ref[...].astype(jnp.bfloat16), z.astype(jnp.bfloat16),
        (((1,), (0,)), ((), ())),
        preferred_element_type=jnp.float32)
    t = jnp.tanh(s + b_ref[0]).astype(jnp.bfloat16)
    o_ref[...] = z + jax.lax.dot_general(
        u_ref[...].astype(jnp.bfloat16), t,
        (((0,), (0,)), ((), ())),
        preferred_element_type=jnp.float32)


@functools.partial(jax.jit, static_argnames=("tile_lanes",))
def _planar_flow(x, w, u, b, tile_lanes=_TILE_LANES):
    N, d = x.shape
    xt = x.T
    tl = min(tile_lanes, N)
    out_t = pl.pallas_call(
        _pf_t_kernel,
        out_shape=jax.ShapeDtypeStruct((d, N), x.dtype),
        grid=(pl.cdiv(N, tl),),
        in_specs=[
            pl.BlockSpec((d, tl), lambda i: (0, i)),
            pl.BlockSpec((1, d), lambda i: (0, 0)),
            pl.BlockSpec((1, d), lambda i: (0, 0)),
            pl.BlockSpec(memory_space=pltpu.MemorySpace.SMEM),
        ],
        out_specs=pl.BlockSpec((d, tl), lambda i: (0, i)),
        compiler_params=pltpu.CompilerParams(
            dimension_semantics=("parallel",),
        ),
    )(xt, w.reshape(1, d), u.reshape(1, d), b.reshape(1))
    return out_t.T


def kernel(x, w, u, b):
    return _planar_flow(x, w, u, b)


# speedup vs baseline: 1.0291x; 1.0291x over previous
import functools

import jax
import jax.numpy as jnp
from jax.experimental import pallas as pl
from jax.experimental.pallas import tpu as pltpu

_TILE_LANES = 32768


def _pf_t_kernel(xt_ref, w_ref, u_ref, b_ref, o_ref):
    z = xt_ref[...]
    s = jax.lax.dot_general(
        w_ref[...].astype(jnp.bfloat16), z.astype(jnp.bfloat16),
        (((1,), (0,)), ((), ())),
        preferred_element_type=jnp.float32)
    t = jnp.tanh(s + b_ref[0]).astype(jnp.bfloat16)
    o_ref[...] = z + jax.lax.dot_general(
        u_ref[...].astype(jnp.bfloat16), t,
        (((0,), (0,)), ((), ())),
        preferred_element_type=jnp.float32)


@functools.partial(jax.jit, static_argnames=("tile_lanes",))
def _planar_flow(x, w, u, b, tile_lanes=_TILE_LANES):
    N, d = x.shape
    xt = x.T
    tl = min(tile_lanes, N)
    out_t = pl.pallas_call(
        _pf_t_kernel,
        out_shape=jax.ShapeDtypeStruct((d, N), x.dtype),
        grid=(pl.cdiv(N, tl),),
        in_specs=[
            pl.BlockSpec((d, tl), lambda i: (0, i)),
            pl.BlockSpec((1, d), lambda i: (0, 0)),
            pl.BlockSpec((1, d), lambda i: (0, 0)),
            pl.BlockSpec(memory_space=pltpu.MemorySpace.SMEM),
        ],
        out_specs=pl.BlockSpec((d, tl), lambda i: (0, i)),
        compiler_params=pltpu.CompilerParams(
            dimension_semantics=("parallel",),
        ),
    )(xt, w.reshape(1, d), u.reshape(1, d), b.reshape(1))
    return out_t.T


def kernel(x, w, u, b):
    return _planar_flow(x, w, u, b)
